# baseline (speedup 1.0000x reference)
import functools

import jax
import jax.numpy as jnp
from jax import lax
from jax.experimental import pallas as pl
from jax.experimental.pallas import tpu as pltpu
from jax.experimental.pallas import tpu_sc as plsc

N = 10000
E = 320000
D = 128
HID = 256
K = 10
ALPHA = 0.1

NC = 2
NS = 16
NW = NC * NS

CH = 128
T_CH = 80
E_PAD = NW * T_CH * CH
S_CH = 16
NSEC = T_CH // S_CH
N_PAD = 10240
ROWS_PER_TILE = N_PAD // NS
DUMMY = N_PAD - 8

RMLP = 1024
RB = RMLP // 128
GRID = N_PAD // RMLP

_sc_mesh = plsc.VectorSubcoreMesh(core_axis_name="c", subcore_axis_name="s")


def _deg_body(srcr, dstr, ones_h, zeros_h, dpo, dpi, onesv, idxb, dego_sp,
              degi_sp):
  c = lax.axis_index("c")
  s = lax.axis_index("s")
  w = c * NS + s

  pltpu.sync_copy(zeros_h, dego_sp.at[pl.ds(s * ROWS_PER_TILE, ROWS_PER_TILE)])
  pltpu.sync_copy(zeros_h, degi_sp.at[pl.ds(s * ROWS_PER_TILE, ROWS_PER_TILE)])
  pltpu.sync_copy(ones_h, onesv)
  plsc.subcore_barrier()

  pltpu.sync_copy(srcr.at[w], idxb)

  def body_o(j, carry):
    pltpu.sync_copy(onesv, dego_sp.at[idxb.at[j]], add=True)
    return carry

  lax.fori_loop(0, T_CH, body_o, 0)

  pltpu.sync_copy(dstr.at[w], idxb)

  def body_i(j, carry):
    pltpu.sync_copy(onesv, degi_sp.at[idxb.at[j]], add=True)
    return carry

  lax.fori_loop(0, T_CH, body_i, 0)

  plsc.subcore_barrier()
  sl = pl.ds(s * ROWS_PER_TILE, ROWS_PER_TILE)
  pltpu.sync_copy(dego_sp.at[sl], dpo.at[c, sl])
  pltpu.sync_copy(degi_sp.at[sl], dpi.at[c, sl])


_deg_call = functools.partial(
    pl.kernel,
    out_type=(
        jax.ShapeDtypeStruct((NC, N_PAD, 16), jnp.float32),
        jax.ShapeDtypeStruct((NC, N_PAD, 16), jnp.float32),
    ),
    mesh=_sc_mesh,
    scratch_types=[
        pltpu.VMEM((CH, 16), jnp.float32),
        pltpu.VMEM((T_CH, CH), jnp.int32),
        pltpu.VMEM_SHARED((N_PAD, 16), jnp.float32),
        pltpu.VMEM_SHARED((N_PAD, 16), jnp.float32),
    ],
)(_deg_body)


ZROWS = 40


def _scatter_body(x_hbm, srcr, dstr, zeros_h, p_hbm, sidx, didx, rows, zbuf,
                  agg_sp, gsem, isem):
  c = lax.axis_index("c")
  s = lax.axis_index("s")
  w = c * NS + s

  pltpu.sync_copy(zeros_h, zbuf)

  def zero_body(j, carry):
    pltpu.sync_copy(zbuf, agg_sp.at[pl.ds(s * ROWS_PER_TILE + j * ZROWS,
                                          ZROWS)])
    return carry

  lax.fori_loop(0, ROWS_PER_TILE // ZROWS, zero_body, 0)

  sx = (sidx.at[0], sidx.at[1])
  dx = (didx.at[0], didx.at[1])
  rw = (rows.at[0], rows.at[1])
  gs = (gsem.at[0], gsem.at[1])

  def prefetch(sec):
    b = sec % 2
    sl = pl.ds(sec * S_CH, S_CH)
    pltpu.async_copy(srcr.at[w, sl], sx[b], isem.at[b, 0])
    pltpu.async_copy(dstr.at[w, sl], dx[b], isem.at[b, 1])

  def wait_prefetch(sec):
    b = sec % 2
    sl = pl.ds(sec * S_CH, S_CH)
    pltpu.make_async_copy(srcr.at[w, sl], sx[b], isem.at[b, 0]).wait()
    pltpu.make_async_copy(dstr.at[w, sl], dx[b], isem.at[b, 1]).wait()

  def gather(sec, j, b):
    pltpu.async_copy(x_hbm.at[sx[sec % 2].at[j]], rw[b], gs[b])

  def wait_gather(sec, j, b):
    pltpu.make_async_copy(x_hbm.at[sx[sec % 2].at[j]], rw[b], gs[b]).wait()

  def scatter(sec, j, b):
    pltpu.sync_copy(rw[b], agg_sp.at[dx[sec % 2].at[j]], add=True)

  prefetch(0)
  wait_prefetch(0)
  prefetch(1)
  gather(0, 0, 0)
  plsc.subcore_barrier()

  for sec in range(NSEC):
    if 1 <= sec and sec + 1 < NSEC:
      prefetch(sec + 1)

    def body(q, carry):
      j0 = 2 * q
      wait_gather(sec, j0, 0)
      gather(sec, j0 + 1, 1)
      scatter(sec, j0, 0)
      wait_gather(sec, j0 + 1, 1)
      gather(sec, j0 + 2, 0)
      scatter(sec, j0 + 1, 1)
      return carry

    lax.fori_loop(0, S_CH // 2 - 1, body, 0)

    wait_gather(sec, S_CH - 2, 0)
    gather(sec, S_CH - 1, 1)
    scatter(sec, S_CH - 2, 0)
    wait_gather(sec, S_CH - 1, 1)
    if sec + 1 < NSEC:
      wait_prefetch(sec + 1)
      gather(sec + 1, 0, 0)
    scatter(sec, S_CH - 1, 1)

  plsc.subcore_barrier()
  sl = pl.ds(s * ROWS_PER_TILE, ROWS_PER_TILE)
  pltpu.sync_copy(agg_sp.at[sl], p_hbm.at[c, sl])


_scatter_call = functools.partial(
    pl.kernel,
    out_type=jax.ShapeDtypeStruct((NC, N_PAD, D), jnp.float32),
    mesh=_sc_mesh,
    scratch_types=[
        pltpu.VMEM((2, S_CH, CH), jnp.int32),
        pltpu.VMEM((2, S_CH, CH), jnp.int32),
        pltpu.VMEM((2, CH, D), jnp.float32),
        pltpu.VMEM((ZROWS, D), jnp.float32),
        pltpu.VMEM_SHARED((N_PAD, D), jnp.float32),
        pltpu.SemaphoreType.DMA((2,)),
        pltpu.SemaphoreType.DMA((2, 2)),
    ],
)(_scatter_body)


def _mlp_body(f, w1, b1, w2, b2, dob, dib, x0, c9, b9, c10, b10):
  h1 = jnp.maximum(
      jnp.dot(f[...], w1[...], preferred_element_type=jnp.float32) + b1[...],
      0.0)
  h = jnp.dot(h1, w2[...], preferred_element_type=jnp.float32) + b2[...]

  no = lax.rsqrt(jnp.maximum(dob[...], 1.0))
  ni = lax.rsqrt(jnp.maximum(dib[...], 1.0))

  x0[...] = h * no
  c9[...] = (1.0 - ALPHA) * no * ni
  b9[...] = (ALPHA * no) * h
  c10[...] = (1.0 - ALPHA) * ni
  b10[...] = ALPHA * h


def _mlp_call(fp, W1, b1r, W2, b2r, dob, dib):
  full = lambda shape: pl.BlockSpec(shape, lambda i: (0,) * len(shape))
  row_spec = pl.BlockSpec((RMLP, 128), lambda i: (i, 0))
  row_shape = jax.ShapeDtypeStruct((N_PAD, D), jnp.float32)
  return pl.pallas_call(
      _mlp_body,
      grid=(GRID,),
      in_specs=[
          row_spec,
          full((128, HID)),
          full((1, HID)),
          full((HID, 128)),
          full((1, 128)),
          row_spec,
          row_spec,
      ],
      out_specs=[row_spec] * 5,
      out_shape=[row_shape] * 5,
  )(fp, W1, b1r, W2, b2r, dob, dib)


def _upd_body(p, cm, b, x_new):
  x_new[...] = cm[...] * (p[0] + p[1]) + b[...]


def _upd_call(p, cm, b):
  row_spec = pl.BlockSpec((RMLP, 128), lambda i: (i, 0))
  return pl.pallas_call(
      _upd_body,
      grid=(GRID,),
      in_specs=[
          pl.BlockSpec((2, RMLP, 128), lambda i: (0, i, 0)),
          row_spec,
          row_spec,
      ],
      out_specs=row_spec,
      out_shape=jax.ShapeDtypeStruct((N_PAD, D), jnp.float32),
  )(p, cm, b)


@jax.jit
def _run(features, edge_index, W1, b1, W2, b2):
  src = edge_index[0]
  dst = edge_index[1]
  pad = E_PAD - E
  pad_idx = jnp.arange(pad, dtype=jnp.int32)
  pad_dst = N + (pad_idx % (N_PAD - N))
  pad_src = N + ((pad_idx * 7) % (N_PAD - N))
  srcp = jnp.concatenate([src, pad_src]).reshape(NW, T_CH, CH)
  dstp = jnp.concatenate([dst, pad_dst]).reshape(NW, T_CH, CH)

  ones_h = jnp.ones((CH, 16), jnp.float32)
  zeros_row = jnp.zeros((ROWS_PER_TILE, 16), jnp.float32)
  zeros_z = jnp.zeros((ZROWS, D), jnp.float32)

  dpo, dpi = _deg_call(srcp, dstp, ones_h, zeros_row)
  dob = jnp.broadcast_to((dpo[0, :, 0] + dpo[1, :, 0])[:, None], (N_PAD, D))
  dib = jnp.broadcast_to((dpi[0, :, 0] + dpi[1, :, 0])[:, None], (N_PAD, D))

  fp = jnp.pad(features, ((0, N_PAD - N), (0, 0)))
  x0, c9, b9, c10, b10 = _mlp_call(fp, W1, b1.reshape(1, HID), W2,
                                   b2.reshape(1, 128), dob, dib)

  x = x0
  for k in range(K):
    p = _scatter_call(x, srcp, dstp, zeros_z)
    if k < K - 1:
      x = _upd_call(p, c9, b9)
    else:
      x = _upd_call(p, c10, b10)
  return x[:N]


def kernel(features, edge_index, W1, b1, W2, b2):
  return _run(features, edge_index, W1, b1, W2, b2)


# speedup vs baseline: 1.0144x; 1.0144x over previous
import functools

import jax
import jax.numpy as jnp
from jax import lax
from jax.experimental import pallas as pl
from jax.experimental.pallas import tpu as pltpu
from jax.experimental.pallas import tpu_sc as plsc

N = 10000
E = 320000
D = 128
HID = 256
K = 10
ALPHA = 0.1

NC = 2
NS = 16
NW = NC * NS

CH = 128
T_CH = 80
E_PAD = NW * T_CH * CH
S_CH = 16
NSEC = T_CH // S_CH
N_PAD = 10240
ROWS_PER_TILE = N_PAD // NS
DUMMY = N_PAD - 8

RMLP = 1024
RB = RMLP // 128
GRID = N_PAD // RMLP

_sc_mesh = plsc.VectorSubcoreMesh(core_axis_name="c", subcore_axis_name="s")


def _deg_body(srcr, dstr, ones_h, zeros_h, dpo, dpi, onesv, idxb, dego_sp,
              degi_sp):
  c = lax.axis_index("c")
  s = lax.axis_index("s")
  w = c * NS + s

  pltpu.sync_copy(zeros_h, dego_sp.at[pl.ds(s * ROWS_PER_TILE, ROWS_PER_TILE)])
  pltpu.sync_copy(zeros_h, degi_sp.at[pl.ds(s * ROWS_PER_TILE, ROWS_PER_TILE)])
  pltpu.sync_copy(ones_h, onesv)
  plsc.subcore_barrier()

  pltpu.sync_copy(srcr.at[w], idxb)

  def body_o(j, carry):
    pltpu.sync_copy(onesv, dego_sp.at[idxb.at[j]], add=True)
    return carry

  lax.fori_loop(0, T_CH, body_o, 0)

  pltpu.sync_copy(dstr.at[w], idxb)

  def body_i(j, carry):
    pltpu.sync_copy(onesv, degi_sp.at[idxb.at[j]], add=True)
    return carry

  lax.fori_loop(0, T_CH, body_i, 0)

  plsc.subcore_barrier()
  sl = pl.ds(s * ROWS_PER_TILE, ROWS_PER_TILE)
  pltpu.sync_copy(dego_sp.at[sl], dpo.at[c, sl])
  pltpu.sync_copy(degi_sp.at[sl], dpi.at[c, sl])


_deg_call = functools.partial(
    pl.kernel,
    out_type=(
        jax.ShapeDtypeStruct((NC, N_PAD), jnp.float32),
        jax.ShapeDtypeStruct((NC, N_PAD), jnp.float32),
    ),
    mesh=_sc_mesh,
    scratch_types=[
        pltpu.VMEM((CH,), jnp.float32),
        pltpu.VMEM((T_CH, CH), jnp.int32),
        pltpu.VMEM_SHARED((N_PAD,), jnp.float32),
        pltpu.VMEM_SHARED((N_PAD,), jnp.float32),
    ],
)(_deg_body)


ZROWS = 40


def _scatter_body(x_hbm, srcr, dstr, zeros_h, p_hbm, sidx, didx, rows, zbuf,
                  agg_sp, gsem, isem):
  c = lax.axis_index("c")
  s = lax.axis_index("s")
  w = c * NS + s

  pltpu.sync_copy(zeros_h, zbuf)

  def zero_body(j, carry):
    pltpu.sync_copy(zbuf, agg_sp.at[pl.ds(s * ROWS_PER_TILE + j * ZROWS,
                                          ZROWS)])
    return carry

  lax.fori_loop(0, ROWS_PER_TILE // ZROWS, zero_body, 0)

  sx = (sidx.at[0], sidx.at[1])
  dx = (didx.at[0], didx.at[1])
  rw = (rows.at[0], rows.at[1])
  gs = (gsem.at[0], gsem.at[1])

  def prefetch(sec):
    b = sec % 2
    sl = pl.ds(sec * S_CH, S_CH)
    pltpu.async_copy(srcr.at[w, sl], sx[b], isem.at[b, 0])
    pltpu.async_copy(dstr.at[w, sl], dx[b], isem.at[b, 1])

  def wait_prefetch(sec):
    b = sec % 2
    sl = pl.ds(sec * S_CH, S_CH)
    pltpu.make_async_copy(srcr.at[w, sl], sx[b], isem.at[b, 0]).wait()
    pltpu.make_async_copy(dstr.at[w, sl], dx[b], isem.at[b, 1]).wait()

  def gather(sec, j, b):
    pltpu.async_copy(x_hbm.at[sx[sec % 2].at[j]], rw[b], gs[b])

  def wait_gather(sec, j, b):
    pltpu.make_async_copy(x_hbm.at[sx[sec % 2].at[j]], rw[b], gs[b]).wait()

  def scatter(sec, j, b):
    pltpu.sync_copy(rw[b], agg_sp.at[dx[sec % 2].at[j]], add=True)

  prefetch(0)
  wait_prefetch(0)
  prefetch(1)
  gather(0, 0, 0)
  plsc.subcore_barrier()

  for sec in range(NSEC):
    if 1 <= sec and sec + 1 < NSEC:
      prefetch(sec + 1)

    def body(q, carry):
      j0 = 2 * q
      wait_gather(sec, j0, 0)
      gather(sec, j0 + 1, 1)
      scatter(sec, j0, 0)
      wait_gather(sec, j0 + 1, 1)
      gather(sec, j0 + 2, 0)
      scatter(sec, j0 + 1, 1)
      return carry

    lax.fori_loop(0, S_CH // 2 - 1, body, 0)

    wait_gather(sec, S_CH - 2, 0)
    gather(sec, S_CH - 1, 1)
    scatter(sec, S_CH - 2, 0)
    wait_gather(sec, S_CH - 1, 1)
    if sec + 1 < NSEC:
      wait_prefetch(sec + 1)
      gather(sec + 1, 0, 0)
    scatter(sec, S_CH - 1, 1)

  plsc.subcore_barrier()
  sl = pl.ds(s * ROWS_PER_TILE, ROWS_PER_TILE)
  pltpu.sync_copy(agg_sp.at[sl], p_hbm.at[c, sl])


_scatter_call = functools.partial(
    pl.kernel,
    out_type=jax.ShapeDtypeStruct((NC, N_PAD, D), jnp.float32),
    mesh=_sc_mesh,
    scratch_types=[
        pltpu.VMEM((2, S_CH, CH), jnp.int32),
        pltpu.VMEM((2, S_CH, CH), jnp.int32),
        pltpu.VMEM((2, CH, D), jnp.float32),
        pltpu.VMEM((ZROWS, D), jnp.float32),
        pltpu.VMEM_SHARED((N_PAD, D), jnp.float32),
        pltpu.SemaphoreType.DMA((2,)),
        pltpu.SemaphoreType.DMA((2, 2)),
    ],
)(_scatter_body)


def _mlp_body(f, w1, b1, w2, b2, dob, dib, x0, c9, b9, c10, b10):
  h1 = jnp.maximum(
      jnp.dot(f[...], w1[...], preferred_element_type=jnp.float32) + b1[...],
      0.0)
  h = jnp.dot(h1, w2[...], preferred_element_type=jnp.float32) + b2[...]

  no = lax.rsqrt(jnp.maximum(dob[...], 1.0))
  ni = lax.rsqrt(jnp.maximum(dib[...], 1.0))

  x0[...] = h * no
  c9[...] = (1.0 - ALPHA) * no * ni
  b9[...] = (ALPHA * no) * h
  c10[...] = (1.0 - ALPHA) * ni
  b10[...] = ALPHA * h


def _mlp_call(fp, W1, b1r, W2, b2r, dob, dib):
  full = lambda shape: pl.BlockSpec(shape, lambda i: (0,) * len(shape))
  row_spec = pl.BlockSpec((RMLP, 128), lambda i: (i, 0))
  row_shape = jax.ShapeDtypeStruct((N_PAD, D), jnp.float32)
  return pl.pallas_call(
      _mlp_body,
      grid=(GRID,),
      in_specs=[
          row_spec,
          full((128, HID)),
          full((1, HID)),
          full((HID, 128)),
          full((1, 128)),
          row_spec,
          row_spec,
      ],
      out_specs=[row_spec] * 5,
      out_shape=[row_shape] * 5,
  )(fp, W1, b1r, W2, b2r, dob, dib)


def _upd_body(p, cm, b, x_new):
  x_new[...] = cm[...] * (p[0] + p[1]) + b[...]


def _upd_call(p, cm, b):
  row_spec = pl.BlockSpec((RMLP, 128), lambda i: (i, 0))
  return pl.pallas_call(
      _upd_body,
      grid=(GRID,),
      in_specs=[
          pl.BlockSpec((2, RMLP, 128), lambda i: (0, i, 0)),
          row_spec,
          row_spec,
      ],
      out_specs=row_spec,
      out_shape=jax.ShapeDtypeStruct((N_PAD, D), jnp.float32),
  )(p, cm, b)


@jax.jit
def _run(features, edge_index, W1, b1, W2, b2):
  src = edge_index[0]
  dst = edge_index[1]
  pad = E_PAD - E
  pad_idx = jnp.arange(pad, dtype=jnp.int32)
  pad_dst = N + (pad_idx % (N_PAD - N))
  pad_src = N + ((pad_idx * 7) % (N_PAD - N))
  srcp = jnp.concatenate([src, pad_src]).reshape(NW, T_CH, CH)
  dstp = jnp.concatenate([dst, pad_dst]).reshape(NW, T_CH, CH)

  ones_h = jnp.ones((CH,), jnp.float32)
  zeros_row = jnp.zeros((ROWS_PER_TILE,), jnp.float32)
  zeros_z = jnp.zeros((ZROWS, D), jnp.float32)

  dpo, dpi = _deg_call(srcp, dstp, ones_h, zeros_row)
  dob = jnp.broadcast_to((dpo[0] + dpo[1])[:, None], (N_PAD, D))
  dib = jnp.broadcast_to((dpi[0] + dpi[1])[:, None], (N_PAD, D))

  fp = jnp.pad(features, ((0, N_PAD - N), (0, 0)))
  x0, c9, b9, c10, b10 = _mlp_call(fp, W1, b1.reshape(1, HID), W2,
                                   b2.reshape(1, 128), dob, dib)

  x = x0
  for k in range(K):
    p = _scatter_call(x, srcp, dstp, zeros_z)
    if k < K - 1:
      x = _upd_call(p, c9, b9)
    else:
      x = _upd_call(p, c10, b10)
  return x[:N]


def kernel(features, edge_index, W1, b1, W2, b2):
  return _run(features, edge_index, W1, b1, W2, b2)


# speedup vs baseline: 1.0294x; 1.0147x over previous
import functools

import jax
import jax.numpy as jnp
from jax import lax
from jax.experimental import pallas as pl
from jax.experimental.pallas import tpu as pltpu
from jax.experimental.pallas import tpu_sc as plsc

N = 10000
E = 320000
D = 128
HID = 256
K = 10
ALPHA = 0.1

NC = 2
NS = 16
NW = NC * NS

CH = 128
T_CH = 80
E_PAD = NW * T_CH * CH
S_CH = 16
NSEC = T_CH // S_CH
N_PAD = 10240
ROWS_PER_TILE = N_PAD // NS
DUMMY = N_PAD - 8

RMLP = 1024
RB = RMLP // 128
GRID = N_PAD // RMLP

_sc_mesh = plsc.VectorSubcoreMesh(core_axis_name="c", subcore_axis_name="s")


def _deg_body(srcr, dstr, ones_h, zeros_h, dpo, dpi, onesv, idxb, dego_sp,
              degi_sp, dsem):
  c = lax.axis_index("c")
  s = lax.axis_index("s")
  w = c * NS + s

  pltpu.sync_copy(zeros_h, dego_sp.at[pl.ds(s * ROWS_PER_TILE, ROWS_PER_TILE)])
  pltpu.sync_copy(zeros_h, degi_sp.at[pl.ds(s * ROWS_PER_TILE, ROWS_PER_TILE)])
  pltpu.sync_copy(ones_h, onesv)
  plsc.subcore_barrier()

  def ring_scatter(acc_sp):
    def issue(j):
      pltpu.async_copy(onesv, acc_sp.at[idxb.at[j]], dsem.at[j % 4],
                       add=True)

    def drain(j):
      pltpu.make_async_copy(onesv, acc_sp.at[idxb.at[j]],
                            dsem.at[j % 4]).wait()

    def body(j, carry):
      issue(j)

      @pl.when(j >= 3)
      def _():
        drain(j - 3)

      return carry

    lax.fori_loop(0, T_CH, body, 0)
    for t in range(3):
      pltpu.make_async_copy(onesv, acc_sp.at[idxb.at[T_CH - 3 + t]],
                            dsem.at[(T_CH - 3 + t) % 4]).wait()

  pltpu.sync_copy(srcr.at[w], idxb)
  ring_scatter(dego_sp)
  pltpu.sync_copy(dstr.at[w], idxb)
  ring_scatter(degi_sp)

  plsc.subcore_barrier()
  sl = pl.ds(s * ROWS_PER_TILE, ROWS_PER_TILE)
  pltpu.sync_copy(dego_sp.at[sl], dpo.at[c, sl])
  pltpu.sync_copy(degi_sp.at[sl], dpi.at[c, sl])


_deg_call = functools.partial(
    pl.kernel,
    out_type=(
        jax.ShapeDtypeStruct((NC, N_PAD), jnp.float32),
        jax.ShapeDtypeStruct((NC, N_PAD), jnp.float32),
    ),
    mesh=_sc_mesh,
    scratch_types=[
        pltpu.VMEM((CH,), jnp.float32),
        pltpu.VMEM((T_CH, CH), jnp.int32),
        pltpu.VMEM_SHARED((N_PAD,), jnp.float32),
        pltpu.VMEM_SHARED((N_PAD,), jnp.float32),
        pltpu.SemaphoreType.DMA((4,)),
    ],
)(_deg_body)


ZROWS = 16


def _scatter_body(x_hbm, srcr, dstr, zeros_h, p_hbm, sidx, didx, rows, zbuf,
                  agg_sp, gsem, ssem, isem):
  c = lax.axis_index("c")
  s = lax.axis_index("s")
  w = c * NS + s

  pltpu.sync_copy(zeros_h, zbuf)

  def zero_body(j, carry):
    pltpu.sync_copy(zbuf, agg_sp.at[pl.ds(s * ROWS_PER_TILE + j * ZROWS,
                                          ZROWS)])
    return carry

  lax.fori_loop(0, ROWS_PER_TILE // ZROWS, zero_body, 0)

  def prefetch(sec):
    b = sec % 2
    sl = pl.ds(sec * S_CH, S_CH)
    pltpu.async_copy(srcr.at[w, sl], sidx.at[b], isem.at[b, 0])
    pltpu.async_copy(dstr.at[w, sl], didx.at[b], isem.at[b, 1])

  def wait_prefetch(sec):
    b = sec % 2
    sl = pl.ds(sec * S_CH, S_CH)
    pltpu.make_async_copy(srcr.at[w, sl], sidx.at[b], isem.at[b, 0]).wait()
    pltpu.make_async_copy(dstr.at[w, sl], didx.at[b], isem.at[b, 1]).wait()

  def s_idx(j):
    return sidx.at[(j // S_CH) % 2].at[j % S_CH]

  def d_idx(j):
    return didx.at[(j // S_CH) % 2].at[j % S_CH]

  def gather(j):
    pltpu.async_copy(x_hbm.at[s_idx(j)], rows.at[j % 2], gsem.at[j % 2])

  def wait_gather(j):
    pltpu.make_async_copy(x_hbm.at[s_idx(j)], rows.at[j % 2],
                          gsem.at[j % 2]).wait()

  def scatter(j):
    pltpu.async_copy(rows.at[j % 2], agg_sp.at[d_idx(j)], ssem.at[j % 2],
                     add=True)

  def wait_scatter(j):
    pltpu.make_async_copy(rows.at[j % 2], agg_sp.at[d_idx(j)],
                          ssem.at[j % 2]).wait()

  prefetch(0)
  wait_prefetch(0)
  prefetch(1)
  gather(0)
  plsc.subcore_barrier()

  def body(j, carry):
    wait_gather(j)
    scatter(j)

    @pl.when(j >= 1)
    def _():
      wait_scatter(j - 1)

    @pl.when(
        jnp.logical_and(j % S_CH == 0,
                        jnp.logical_and(j >= S_CH,
                                        j // S_CH + 1 < NSEC)))
    def _():
      prefetch(j // S_CH + 1)

    @pl.when(j + 1 < T_CH)
    def _():
      @pl.when((j + 1) % S_CH == 0)
      def _():
        wait_prefetch(j // S_CH + 1)

      gather(j + 1)

    return carry

  lax.fori_loop(0, T_CH, body, 0)
  wait_scatter(T_CH - 1)

  plsc.subcore_barrier()
  sl = pl.ds(s * ROWS_PER_TILE, ROWS_PER_TILE)
  pltpu.sync_copy(agg_sp.at[sl], p_hbm.at[c, sl])


_scatter_call = functools.partial(
    pl.kernel,
    out_type=jax.ShapeDtypeStruct((NC, N_PAD, D), jnp.float32),
    mesh=_sc_mesh,
    scratch_types=[
        pltpu.VMEM((2, S_CH, CH), jnp.int32),
        pltpu.VMEM((2, S_CH, CH), jnp.int32),
        pltpu.VMEM((2, CH, D), jnp.float32),
        pltpu.VMEM((ZROWS, D), jnp.float32),
        pltpu.VMEM_SHARED((N_PAD, D), jnp.float32),
        pltpu.SemaphoreType.DMA((2,)),
        pltpu.SemaphoreType.DMA((2,)),
        pltpu.SemaphoreType.DMA((2, 2)),
    ],
)(_scatter_body)


def _mlp_body(f, w1, b1, w2, b2, dob, dib, x0, c9, b9, c10, b10):
  h1 = jnp.maximum(
      jnp.dot(f[...], w1[...], preferred_element_type=jnp.float32) + b1[...],
      0.0)
  h = jnp.dot(h1, w2[...], preferred_element_type=jnp.float32) + b2[...]

  no = lax.rsqrt(jnp.maximum(dob[...], 1.0))
  ni = lax.rsqrt(jnp.maximum(dib[...], 1.0))

  x0[...] = h * no
  c9[...] = (1.0 - ALPHA) * no * ni
  b9[...] = (ALPHA * no) * h
  c10[...] = (1.0 - ALPHA) * ni
  b10[...] = ALPHA * h


def _mlp_call(fp, W1, b1r, W2, b2r, dob, dib):
  full = lambda shape: pl.BlockSpec(shape, lambda i: (0,) * len(shape))
  row_spec = pl.BlockSpec((RMLP, 128), lambda i: (i, 0))
  row_shape = jax.ShapeDtypeStruct((N_PAD, D), jnp.float32)
  return pl.pallas_call(
      _mlp_body,
      grid=(GRID,),
      in_specs=[
          row_spec,
          full((128, HID)),
          full((1, HID)),
          full((HID, 128)),
          full((1, 128)),
          row_spec,
          row_spec,
      ],
      out_specs=[row_spec] * 5,
      out_shape=[row_shape] * 5,
  )(fp, W1, b1r, W2, b2r, dob, dib)


def _upd_body(p, cm, b, x_new):
  x_new[...] = cm[...] * (p[0] + p[1]) + b[...]


def _upd_call(p, cm, b):
  row_spec = pl.BlockSpec((RMLP, 128), lambda i: (i, 0))
  return pl.pallas_call(
      _upd_body,
      grid=(GRID,),
      in_specs=[
          pl.BlockSpec((2, RMLP, 128), lambda i: (0, i, 0)),
          row_spec,
          row_spec,
      ],
      out_specs=row_spec,
      out_shape=jax.ShapeDtypeStruct((N_PAD, D), jnp.float32),
  )(p, cm, b)


@jax.jit
def _run(features, edge_index, W1, b1, W2, b2):
  src = edge_index[0]
  dst = edge_index[1]
  pad = E_PAD - E
  pad_idx = jnp.arange(pad, dtype=jnp.int32)
  pad_dst = N + (pad_idx % (N_PAD - N))
  pad_src = N + ((pad_idx * 7) % (N_PAD - N))
  srcp = jnp.concatenate([src, pad_src]).reshape(NW, T_CH, CH)
  dstp = jnp.concatenate([dst, pad_dst]).reshape(NW, T_CH, CH)

  ones_h = jnp.ones((CH,), jnp.float32)
  zeros_row = jnp.zeros((ROWS_PER_TILE,), jnp.float32)
  zeros_z = jnp.zeros((ZROWS, D), jnp.float32)

  dpo, dpi = _deg_call(srcp, dstp, ones_h, zeros_row)
  dob = jnp.broadcast_to((dpo[0] + dpo[1])[:, None], (N_PAD, D))
  dib = jnp.broadcast_to((dpi[0] + dpi[1])[:, None], (N_PAD, D))

  fp = jnp.pad(features, ((0, N_PAD - N), (0, 0)))
  x0, c9, b9, c10, b10 = _mlp_call(fp, W1, b1.reshape(1, HID), W2,
                                   b2.reshape(1, 128), dob, dib)

  x = x0
  for k in range(K):
    p = _scatter_call(x, srcp, dstp, zeros_z)
    if k < K - 1:
      x = _upd_call(p, c9, b9)
    else:
      x = _upd_call(p, c10, b10)
  return x[:N]


def kernel(features, edge_index, W1, b1, W2, b2):
  return _run(features, edge_index, W1, b1, W2, b2)


# speedup vs baseline: 1.0602x; 1.0299x over previous
import functools

import jax
import jax.numpy as jnp
from jax import lax
from jax.experimental import pallas as pl
from jax.experimental.pallas import tpu as pltpu
from jax.experimental.pallas import tpu_sc as plsc

N = 10000
E = 320000
D = 128
HID = 256
K = 10
ALPHA = 0.1

NC = 2
NS = 16
NW = NC * NS

CH = 128
T_CH = 80
E_PAD = NW * T_CH * CH
S_CH = 16
NSEC = T_CH // S_CH
N_PAD = 10240
ROWS_PER_TILE = N_PAD // NS
DUMMY = N_PAD - 8

RMLP = 1024
RB = RMLP // 128
GRID = N_PAD // RMLP

_sc_mesh = plsc.VectorSubcoreMesh(core_axis_name="c", subcore_axis_name="s")


def _deg_body(srcr, dstr, ones_h, zeros_h, dpo, dpi, onesv, idxb, dego_sp,
              degi_sp, dsem):
  c = lax.axis_index("c")
  s = lax.axis_index("s")
  w = c * NS + s

  pltpu.sync_copy(zeros_h, dego_sp.at[pl.ds(s * ROWS_PER_TILE, ROWS_PER_TILE)])
  pltpu.sync_copy(zeros_h, degi_sp.at[pl.ds(s * ROWS_PER_TILE, ROWS_PER_TILE)])
  pltpu.sync_copy(ones_h, onesv)
  plsc.subcore_barrier()

  def ring_scatter(acc_sp):
    def issue(j):
      pltpu.async_copy(onesv, acc_sp.at[idxb.at[j]], dsem.at[j % 4],
                       add=True)

    def drain(j):
      pltpu.make_async_copy(onesv, acc_sp.at[idxb.at[j]],
                            dsem.at[j % 4]).wait()

    def body(j, carry):
      issue(j)

      @pl.when(j >= 3)
      def _():
        drain(j - 3)

      return carry

    lax.fori_loop(0, T_CH, body, 0)
    for t in range(3):
      pltpu.make_async_copy(onesv, acc_sp.at[idxb.at[T_CH - 3 + t]],
                            dsem.at[(T_CH - 3 + t) % 4]).wait()

  pltpu.sync_copy(srcr.at[w], idxb)
  ring_scatter(dego_sp)
  pltpu.sync_copy(dstr.at[w], idxb)
  ring_scatter(degi_sp)

  plsc.subcore_barrier()
  sl = pl.ds(s * ROWS_PER_TILE, ROWS_PER_TILE)
  pltpu.sync_copy(dego_sp.at[sl], dpo.at[c, sl])
  pltpu.sync_copy(degi_sp.at[sl], dpi.at[c, sl])


_deg_call = functools.partial(
    pl.kernel,
    out_type=(
        jax.ShapeDtypeStruct((NC, N_PAD), jnp.float32),
        jax.ShapeDtypeStruct((NC, N_PAD), jnp.float32),
    ),
    mesh=_sc_mesh,
    scratch_types=[
        pltpu.VMEM((CH,), jnp.float32),
        pltpu.VMEM((T_CH, CH), jnp.int32),
        pltpu.VMEM_SHARED((N_PAD,), jnp.float32),
        pltpu.VMEM_SHARED((N_PAD,), jnp.float32),
        pltpu.SemaphoreType.DMA((4,)),
    ],
)(_deg_body)


ZROWS = 16


def _scatter_body(x_hbm, srcr, dstr, zeros_h, p_hbm, sidx, didx, rows, zbuf,
                  agg_sp, gsem, ssem, isem, zsem):
  c = lax.axis_index("c")
  s = lax.axis_index("s")
  w = c * NS + s

  pltpu.sync_copy(zeros_h, zbuf)

  def zero_issue(j, carry):
    pltpu.async_copy(zbuf, agg_sp.at[pl.ds(s * ROWS_PER_TILE + j * ZROWS,
                                           ZROWS)], zsem)
    return carry

  lax.fori_loop(0, ROWS_PER_TILE // ZROWS, zero_issue, 0)

  def zero_drain(j, carry):
    pltpu.make_async_copy(zbuf, agg_sp.at[pl.ds(s * ROWS_PER_TILE + j * ZROWS,
                                                ZROWS)], zsem).wait()
    return carry

  def prefetch(sec):
    b = sec % 2
    sl = pl.ds(sec * S_CH, S_CH)
    pltpu.async_copy(srcr.at[w, sl], sidx.at[b], isem.at[b, 0])
    pltpu.async_copy(dstr.at[w, sl], didx.at[b], isem.at[b, 1])

  def wait_prefetch(sec):
    b = sec % 2
    sl = pl.ds(sec * S_CH, S_CH)
    pltpu.make_async_copy(srcr.at[w, sl], sidx.at[b], isem.at[b, 0]).wait()
    pltpu.make_async_copy(dstr.at[w, sl], didx.at[b], isem.at[b, 1]).wait()

  def s_idx(j):
    return sidx.at[(j // S_CH) % 2].at[j % S_CH]

  def d_idx(j):
    return didx.at[(j // S_CH) % 2].at[j % S_CH]

  def gather(j):
    pltpu.async_copy(x_hbm.at[s_idx(j)], rows.at[j % 2], gsem.at[j % 2])

  def wait_gather(j):
    pltpu.make_async_copy(x_hbm.at[s_idx(j)], rows.at[j % 2],
                          gsem.at[j % 2]).wait()

  def scatter(j):
    pltpu.async_copy(rows.at[j % 2], agg_sp.at[d_idx(j)], ssem.at[j % 2],
                     add=True)

  def wait_scatter(j):
    pltpu.make_async_copy(rows.at[j % 2], agg_sp.at[d_idx(j)],
                          ssem.at[j % 2]).wait()

  prefetch(0)
  wait_prefetch(0)
  prefetch(1)
  gather(0)
  lax.fori_loop(0, ROWS_PER_TILE // ZROWS, zero_drain, 0)
  plsc.subcore_barrier()

  def body(j, carry):
    wait_gather(j)
    scatter(j)

    @pl.when(j >= 1)
    def _():
      wait_scatter(j - 1)

    @pl.when(
        jnp.logical_and(j % S_CH == 0,
                        jnp.logical_and(j >= S_CH,
                                        j // S_CH + 1 < NSEC)))
    def _():
      prefetch(j // S_CH + 1)

    @pl.when(j + 1 < T_CH)
    def _():
      @pl.when((j + 1) % S_CH == 0)
      def _():
        wait_prefetch(j // S_CH + 1)

      gather(j + 1)

    return carry

  lax.fori_loop(0, T_CH, body, 0)
  wait_scatter(T_CH - 1)

  plsc.subcore_barrier()
  sl = pl.ds(s * ROWS_PER_TILE, ROWS_PER_TILE)
  pltpu.sync_copy(agg_sp.at[sl], p_hbm.at[c, sl])


_scatter_call = functools.partial(
    pl.kernel,
    out_type=jax.ShapeDtypeStruct((NC, N_PAD, D), jnp.float32),
    mesh=_sc_mesh,
    scratch_types=[
        pltpu.VMEM((2, S_CH, CH), jnp.int32),
        pltpu.VMEM((2, S_CH, CH), jnp.int32),
        pltpu.VMEM((2, CH, D), jnp.float32),
        pltpu.VMEM((ZROWS, D), jnp.float32),
        pltpu.VMEM_SHARED((N_PAD, D), jnp.float32),
        pltpu.SemaphoreType.DMA((2,)),
        pltpu.SemaphoreType.DMA((2,)),
        pltpu.SemaphoreType.DMA((2, 2)),
        pltpu.SemaphoreType.DMA,
    ],
)(_scatter_body)


def _mlp_body(f, w1, b1, w2, b2, dob, dib, x0, c9, b9, c10, b10):
  h1 = jnp.maximum(
      jnp.dot(f[...], w1[...], preferred_element_type=jnp.float32) + b1[...],
      0.0)
  h = jnp.dot(h1, w2[...], preferred_element_type=jnp.float32) + b2[...]

  no = lax.rsqrt(jnp.maximum(dob[...], 1.0))
  ni = lax.rsqrt(jnp.maximum(dib[...], 1.0))

  x0[...] = h * no
  c9[...] = (1.0 - ALPHA) * no * ni
  b9[...] = (ALPHA * no) * h
  c10[...] = (1.0 - ALPHA) * ni
  b10[...] = ALPHA * h


def _mlp_call(fp, W1, b1r, W2, b2r, dob, dib):
  full = lambda shape: pl.BlockSpec(shape, lambda i: (0,) * len(shape))
  row_spec = pl.BlockSpec((RMLP, 128), lambda i: (i, 0))
  row_shape = jax.ShapeDtypeStruct((N_PAD, D), jnp.float32)
  return pl.pallas_call(
      _mlp_body,
      grid=(GRID,),
      in_specs=[
          row_spec,
          full((128, HID)),
          full((1, HID)),
          full((HID, 128)),
          full((1, 128)),
          row_spec,
          row_spec,
      ],
      out_specs=[row_spec] * 5,
      out_shape=[row_shape] * 5,
  )(fp, W1, b1r, W2, b2r, dob, dib)


def _upd_body(p, cm, b, x_new):
  x_new[...] = cm[...] * (p[0] + p[1]) + b[...]


def _upd_call(p, cm, b):
  row_spec = pl.BlockSpec((RMLP, 128), lambda i: (i, 0))
  return pl.pallas_call(
      _upd_body,
      grid=(GRID,),
      in_specs=[
          pl.BlockSpec((2, RMLP, 128), lambda i: (0, i, 0)),
          row_spec,
          row_spec,
      ],
      out_specs=row_spec,
      out_shape=jax.ShapeDtypeStruct((N_PAD, D), jnp.float32),
  )(p, cm, b)


@jax.jit
def _run(features, edge_index, W1, b1, W2, b2):
  src = edge_index[0]
  dst = edge_index[1]
  pad = E_PAD - E
  pad_idx = jnp.arange(pad, dtype=jnp.int32)
  pad_dst = N + (pad_idx % (N_PAD - N))
  pad_src = N + ((pad_idx * 7) % (N_PAD - N))
  srcp = jnp.concatenate([src, pad_src]).reshape(NW, T_CH, CH)
  dstp = jnp.concatenate([dst, pad_dst]).reshape(NW, T_CH, CH)

  ones_h = jnp.ones((CH,), jnp.float32)
  zeros_row = jnp.zeros((ROWS_PER_TILE,), jnp.float32)
  zeros_z = jnp.zeros((ZROWS, D), jnp.float32)

  dpo, dpi = _deg_call(srcp, dstp, ones_h, zeros_row)
  dob = jnp.broadcast_to((dpo[0] + dpo[1])[:, None], (N_PAD, D))
  dib = jnp.broadcast_to((dpi[0] + dpi[1])[:, None], (N_PAD, D))

  fp = jnp.pad(features, ((0, N_PAD - N), (0, 0)))
  x0, c9, b9, c10, b10 = _mlp_call(fp, W1, b1.reshape(1, HID), W2,
                                   b2.reshape(1, 128), dob, dib)

  x = x0
  for k in range(K):
    p = _scatter_call(x, srcp, dstp, zeros_z)
    if k < K - 1:
      x = _upd_call(p, c9, b9)
    else:
      x = _upd_call(p, c10, b10)
  return x[:N]


def kernel(features, edge_index, W1, b1, W2, b2):
  return _run(features, edge_index, W1, b1, W2, b2)
